# baseline (speedup 1.0000x reference)
import functools

import jax
import jax.numpy as jnp
from jax.experimental import pallas as pl
from jax.experimental.pallas import tpu as pltpu
from jax.experimental.pallas import tpu_sc as plsc

N = 2048
E = 65536
H = 128
K = 512
F_IN = 128

_NS = 16
_EPT = E // _NS
_QROWS = N // 4
_QS = _QROWS * N
_SPT = _QS // _NS
_ZCH = 4096


def _st_body(edges, st_out, srcv, dstv, idxa, idxb, onesv, zbuf, acc, sem):
    s = jax.lax.axis_index("s")
    c = jax.lax.axis_index("c")

    def fill_ones(k, _):
        onesv[pl.ds(k * 16, 16)] = jnp.full((16,), 1.0, jnp.float32)
        return 0
    jax.lax.fori_loop(0, _EPT // 16, fill_ones, 0)

    def fill_z(k, _):
        zbuf[pl.ds(k * 16, 16)] = jnp.zeros((16,), jnp.float32)
        return 0
    jax.lax.fori_loop(0, _ZCH // 16, fill_z, 0)

    base_e = s * _EPT
    pltpu.sync_copy(edges.at[0, pl.ds(base_e, _EPT)], srcv)
    pltpu.sync_copy(edges.at[1, pl.ds(base_e, _EPT)], dstv)

    zc = [pltpu.async_copy(zbuf, acc.at[pl.ds(s * _SPT + t * _ZCH, _ZCH)], sem)
          for t in range(_SPT // _ZCH)]

    qlo_a = (c * 2) * _QROWS
    qlo_b = (c * 2 + 1) * _QROWS

    def comp(j, _):
        def inner(k, _):
            off = j * 128 + k * 16
            sv = srcv[pl.ds(off, 16)]
            dv = dstv[pl.ds(off, 16)]
            trash = _QS + k * 16 + jax.lax.iota(jnp.int32, 16)
            wa = (dv - qlo_a) * N + sv
            va = (dv >= qlo_a) & (dv < qlo_a + _QROWS)
            idxa[pl.ds(off, 16)] = jnp.where(va, wa, trash)
            wb = (dv - qlo_b) * N + sv
            vb = (dv >= qlo_b) & (dv < qlo_b + _QROWS)
            idxb[pl.ds(off, 16)] = jnp.where(vb, wb, trash)
            return 0
        jax.lax.fori_loop(0, 8, inner, 0)
        return 0
    jax.lax.fori_loop(0, _EPT // 128, comp, 0)

    for cp in zc:
        cp.wait()

    for q_local, idxv in ((0, idxa), (1, idxb)):
        q = c * 2 + q_local
        plsc.subcore_barrier()
        pltpu.async_copy(onesv, acc.at[idxv], sem, add=True).wait()
        plsc.subcore_barrier()
        pltpu.sync_copy(acc.at[pl.ds(s * _SPT, _SPT)],
                        st_out.at[pl.ds(q * _QS + s * _SPT, _SPT)])
        if q_local == 0:
            zc2 = [pltpu.async_copy(
                zbuf, acc.at[pl.ds(s * _SPT + t * _ZCH, _ZCH)], sem)
                for t in range(_SPT // _ZCH)]
            for cp in zc2:
                cp.wait()


@functools.cache
def _build_st_fn():
    return pl.kernel(
        _st_body,
        out_type=jax.ShapeDtypeStruct((N * N,), jnp.float32),
        mesh=plsc.VectorSubcoreMesh(core_axis_name="c", subcore_axis_name="s"),
        scratch_types=[
            pltpu.VMEM((_EPT,), jnp.int32),
            pltpu.VMEM((_EPT,), jnp.int32),
            pltpu.VMEM((_EPT,), jnp.int32),
            pltpu.VMEM((_EPT,), jnp.int32),
            pltpu.VMEM((_EPT,), jnp.float32),
            pltpu.VMEM((_ZCH,), jnp.float32),
            pltpu.VMEM_SHARED((_QS + 128,), jnp.float32),
            pltpu.SemaphoreType.DMA,
        ],
    )


def _dense_body(nodes_ref, st_ref, w1_ref, b1_ref, w2_ref, b2_ref, wq_ref,
                wk_ref, wv_ref, seeds_ref, lng_ref, lnb_ref, w3_ref, b3_ref,
                w4_ref, b4_ref, w5_ref, b5_ref, x_out_ref, adj_out_ref):
    f32 = jnp.float32
    st = st_ref[...]
    deg = jnp.sum(st, axis=1, keepdims=True) + 1.0
    dinv = jax.lax.rsqrt(deg)
    dinv2 = dinv * dinv

    def gcn(x, w_ref, b_ref):
        h = jnp.dot(x, w_ref[...], preferred_element_type=f32)
        agg = jnp.dot(st, h * dinv, preferred_element_type=f32)
        return dinv * agg + dinv2 * h + b_ref[...]

    x = jnp.tanh(gcn(nodes_ref[...], w1_ref, b1_ref))
    x = jnp.tanh(gcn(x, w2_ref, b2_ref))

    q = jnp.dot(seeds_ref[...], wq_ref[...], preferred_element_type=f32)
    k_ = jnp.dot(x, wk_ref[...], preferred_element_type=f32)
    v = jnp.dot(x, wv_ref[...], preferred_element_type=f32)
    logits = jax.lax.dot_general(q, k_, (((1,), (1,)), ((), ())),
                                 preferred_element_type=f32)
    logits = logits * (1.0 / jnp.sqrt(jnp.float32(H)))
    m = jnp.max(logits, axis=1, keepdims=True)
    p = jnp.exp(logits - m)
    attn = p / jnp.sum(p, axis=1, keepdims=True)
    pooled = jnp.dot(attn, v, preferred_element_type=f32)
    mu = jnp.mean(pooled, axis=1, keepdims=True)
    var = jnp.mean((pooled - mu) ** 2, axis=1, keepdims=True)
    xp = (pooled - mu) * jax.lax.rsqrt(var + 1e-5) * lng_ref[...] + lnb_ref[...]

    x_mid = jax.lax.dot_general(attn, xp, (((0,), (0,)), ((), ())),
                                preferred_element_type=f32)

    st_bf = st.astype(jnp.bfloat16)
    attn_bf = attn.astype(jnp.bfloat16)
    tmpT = jax.lax.dot_general(st_bf, attn_bf, (((1,), (1,)), ((), ())),
                               preferred_element_type=f32)
    pool_adj = jax.lax.dot_general(tmpT, attn, (((0,), (1,)), ((), ())),
                                   preferred_element_type=f32)
    tmp2 = jnp.dot(pool_adj, attn, preferred_element_type=f32)
    adj_out_ref[...] = jax.lax.dot_general(
        attn_bf, tmp2.astype(jnp.bfloat16), (((0,), (0,)), ((), ())),
        preferred_element_type=f32)

    x = jnp.tanh(gcn(x_mid, w3_ref, b3_ref))
    x = jnp.tanh(gcn(x, w4_ref, b4_ref))
    x_out_ref[...] = gcn(x, w5_ref, b5_ref)


@jax.jit
def _dense_chain(nodes, st, p):
    out_shape = (jax.ShapeDtypeStruct((N, F_IN), jnp.float32),
                 jax.ShapeDtypeStruct((N, N), jnp.float32))
    fn = pl.pallas_call(
        _dense_body,
        out_shape=out_shape,
        compiler_params=pltpu.CompilerParams(
            vmem_limit_bytes=128 * 1024 * 1024),
    )
    return fn(nodes, st,
              p["W1"], p["b1"].reshape(1, H), p["W2"], p["b2"].reshape(1, H),
              p["Wq"], p["Wk"], p["Wv"], p["seeds"],
              p["ln_g"].reshape(1, H), p["ln_b"].reshape(1, H),
              p["W3"], p["b3"].reshape(1, H), p["W4"], p["b4"].reshape(1, H),
              p["W5"], p["b5"].reshape(1, F_IN))


def kernel(nodes, edge_index, batch, params):
    st = _build_st_fn()(edge_index).reshape(N, N)
    x, adj = _dense_chain(nodes, st, params)
    return (x, adj[None])


# speedup vs baseline: 1.0009x; 1.0009x over previous
import functools

import jax
import jax.numpy as jnp
from jax.experimental import pallas as pl
from jax.experimental.pallas import tpu as pltpu
from jax.experimental.pallas import tpu_sc as plsc

N = 2048
E = 65536
H = 128
K = 512
F_IN = 128

_NS = 16
_EPT = E // _NS
_QROWS = N // 4
_QS = _QROWS * N
_SPT = _QS // _NS
_ZCH = 4096


def _st_body(edges, st_out, srcv, dstv, idxa, idxb, onesv, zbuf, acc, sem):
    s = jax.lax.axis_index("s")
    c = jax.lax.axis_index("c")

    def fill_ones(k, _):
        onesv[pl.ds(k * 16, 16)] = jnp.full((16,), 1.0, jnp.float32)
        return 0
    jax.lax.fori_loop(0, _EPT // 16, fill_ones, 0)

    def fill_z(k, _):
        zbuf[pl.ds(k * 16, 16)] = jnp.zeros((16,), jnp.float32)
        return 0
    jax.lax.fori_loop(0, _ZCH // 16, fill_z, 0)

    base_e = s * _EPT
    pltpu.sync_copy(edges.at[0, pl.ds(base_e, _EPT)], srcv)
    pltpu.sync_copy(edges.at[1, pl.ds(base_e, _EPT)], dstv)

    zc = [pltpu.async_copy(zbuf, acc.at[pl.ds(s * _SPT + t * _ZCH, _ZCH)], sem)
          for t in range(_SPT // _ZCH)]

    qlo_a = (c * 2) * _QROWS
    qlo_b = (c * 2 + 1) * _QROWS

    def comp(j, _):
        def inner(k, _):
            off = j * 128 + k * 16
            sv = srcv[pl.ds(off, 16)]
            dv = dstv[pl.ds(off, 16)]
            trash = _QS + k * 16 + jax.lax.iota(jnp.int32, 16)
            wa = (dv - qlo_a) * N + sv
            va = (dv >= qlo_a) & (dv < qlo_a + _QROWS)
            idxa[pl.ds(off, 16)] = jnp.where(va, wa, trash)
            wb = (dv - qlo_b) * N + sv
            vb = (dv >= qlo_b) & (dv < qlo_b + _QROWS)
            idxb[pl.ds(off, 16)] = jnp.where(vb, wb, trash)
            return 0
        jax.lax.fori_loop(0, 8, inner, 0)
        return 0
    jax.lax.fori_loop(0, _EPT // 128, comp, 0)

    for cp in zc:
        cp.wait()

    for q_local, idxv in ((0, idxa), (1, idxb)):
        q = c * 2 + q_local
        plsc.subcore_barrier()
        pltpu.async_copy(onesv, acc.at[idxv], sem, add=True).wait()
        plsc.subcore_barrier()
        pltpu.sync_copy(acc.at[pl.ds(s * _SPT, _SPT)],
                        st_out.at[pl.ds(q * _QS + s * _SPT, _SPT)])
        if q_local == 0:
            zc2 = [pltpu.async_copy(
                zbuf, acc.at[pl.ds(s * _SPT + t * _ZCH, _ZCH)], sem)
                for t in range(_SPT // _ZCH)]
            for cp in zc2:
                cp.wait()


@functools.cache
def _build_st_fn():
    return pl.kernel(
        _st_body,
        out_type=jax.ShapeDtypeStruct((N * N,), jnp.float32),
        mesh=plsc.VectorSubcoreMesh(core_axis_name="c", subcore_axis_name="s"),
        scratch_types=[
            pltpu.VMEM((_EPT,), jnp.int32),
            pltpu.VMEM((_EPT,), jnp.int32),
            pltpu.VMEM((_EPT,), jnp.int32),
            pltpu.VMEM((_EPT,), jnp.int32),
            pltpu.VMEM((_EPT,), jnp.float32),
            pltpu.VMEM((_ZCH,), jnp.float32),
            pltpu.VMEM_SHARED((_QS + 128,), jnp.float32),
            pltpu.SemaphoreType.DMA,
        ],
    )


def _dense_body(nodes_ref, st_ref, w1_ref, b1_ref, w2_ref, b2_ref, wq_ref,
                wk_ref, wv_ref, seeds_ref, lng_ref, lnb_ref, w3_ref, b3_ref,
                w4_ref, b4_ref, w5_ref, b5_ref, x_out_ref, adj_out_ref):
    f32 = jnp.float32
    st = st_ref[...]
    deg = jnp.sum(st, axis=1, keepdims=True) + 1.0
    dinv = jax.lax.rsqrt(deg)
    dinv2 = dinv * dinv

    def gcn(x, w_ref, b_ref):
        h = jnp.dot(x, w_ref[...], preferred_element_type=f32)
        agg = jnp.dot(st, h * dinv, preferred_element_type=f32)
        return dinv * agg + dinv2 * h + b_ref[...]

    x = jnp.tanh(gcn(nodes_ref[...], w1_ref, b1_ref))
    x = jnp.tanh(gcn(x, w2_ref, b2_ref))

    q = jnp.dot(seeds_ref[...], wq_ref[...], preferred_element_type=f32)
    k_ = jnp.dot(x, wk_ref[...], preferred_element_type=f32)
    v = jnp.dot(x, wv_ref[...], preferred_element_type=f32)
    logits = jax.lax.dot_general(q, k_, (((1,), (1,)), ((), ())),
                                 preferred_element_type=f32)
    logits = logits * (1.0 / jnp.sqrt(jnp.float32(H)))
    m = jnp.max(logits, axis=1, keepdims=True)
    p = jnp.exp(logits - m)
    attn = p / jnp.sum(p, axis=1, keepdims=True)
    pooled = jnp.dot(attn, v, preferred_element_type=f32)
    mu = jnp.mean(pooled, axis=1, keepdims=True)
    var = jnp.mean((pooled - mu) ** 2, axis=1, keepdims=True)
    xp = (pooled - mu) * jax.lax.rsqrt(var + 1e-5) * lng_ref[...] + lnb_ref[...]

    x_mid = jax.lax.dot_general(attn, xp, (((0,), (0,)), ((), ())),
                                preferred_element_type=f32)

    tmpT = jax.lax.dot_general(st, attn, (((1,), (1,)), ((), ())),
                               preferred_element_type=f32)
    pool_adj = jax.lax.dot_general(tmpT, attn, (((0,), (1,)), ((), ())),
                                   preferred_element_type=f32)
    tmp2 = jnp.dot(pool_adj, attn, preferred_element_type=f32)
    adj_out_ref[...] = jax.lax.dot_general(
        attn, tmp2, (((0,), (0,)), ((), ())), preferred_element_type=f32)

    x = jnp.tanh(gcn(x_mid, w3_ref, b3_ref))
    x = jnp.tanh(gcn(x, w4_ref, b4_ref))
    x_out_ref[...] = gcn(x, w5_ref, b5_ref)


@jax.jit
def _dense_chain(nodes, st, p):
    out_shape = (jax.ShapeDtypeStruct((N, F_IN), jnp.float32),
                 jax.ShapeDtypeStruct((N, N), jnp.float32))
    fn = pl.pallas_call(
        _dense_body,
        out_shape=out_shape,
        compiler_params=pltpu.CompilerParams(
            vmem_limit_bytes=128 * 1024 * 1024),
    )
    return fn(nodes, st,
              p["W1"], p["b1"].reshape(1, H), p["W2"], p["b2"].reshape(1, H),
              p["Wq"], p["Wk"], p["Wv"], p["seeds"],
              p["ln_g"].reshape(1, H), p["ln_b"].reshape(1, H),
              p["W3"], p["b3"].reshape(1, H), p["W4"], p["b4"].reshape(1, H),
              p["W5"], p["b5"].reshape(1, F_IN))


def kernel(nodes, edge_index, batch, params):
    st = _build_st_fn()(edge_index).reshape(N, N)
    x, adj = _dense_chain(nodes, st, params)
    return (x, adj[None])
